# Initial kernel scaffold; baseline (speedup 1.0000x reference)
import jax
import jax.numpy as jnp
from jax.experimental import pallas as pl


def kernel(feature, adj_new_list, labels, chosen_tail_lists, first_neighbor_lists, second_neighbor_lists, center_dict_lists, W_s, W1, W2):
    raise NotImplementedError("write your pallas kernel here")


# speedup vs baseline: 3.8800x; 3.8800x over previous
import functools

import jax
import jax.numpy as jnp
from jax import lax
from jax.experimental import pallas as pl
from jax.experimental.pallas import tpu as pltpu
from jax.experimental.pallas import tpu_sc as plsc

N = 10000
E = 320000
L = 2
T = 1000
D = 128
H = 128

NC = 2
NS = 16

TPAD = 1024
RD = N + 16
CHUNK = 128
EPT = 20096
EPL = EPT * NS

_mesh = plsc.VectorSubcoreMesh(core_axis_name="c", subcore_axis_name="s")
_sc_params = pltpu.CompilerParams(needs_layout_passes=False)


def _splat_i32(x):
    return jnp.zeros((16,), jnp.int32) + x


@functools.partial(
    pl.kernel,
    out_type=(
        jax.ShapeDtypeStruct((6 * TPAD, D), jnp.float32),
        jax.ShapeDtypeStruct((L * TPAD,), jnp.int32),
    ),
    mesh=_mesh,
    scratch_types=[
        pltpu.VMEM((96,), jnp.int32),
        pltpu.VMEM((96, D), jnp.float32),
        pltpu.VMEM((TPAD,), jnp.int32),
        pltpu.VMEM((64,), jnp.int32),
        pltpu.SemaphoreType.DMA,
    ],
    compiler_params=_sc_params,
)
def _k1(feat_hbm, gidx_hbm, chosen_hbm, rows_hbm, sidx_hbm,
        idx_v, rows_v, ch_v, sidx_v, sem):
    c = lax.axis_index("c")
    s = lax.axis_index("s")
    wid = c * NS + s
    for j in range(2):
        base = wid * 192 + j * 96
        pltpu.sync_copy(gidx_hbm.at[pl.ds(base, 96)], idx_v)
        pltpu.async_copy(feat_hbm.at[idx_v], rows_v, sem).wait()
        pltpu.sync_copy(rows_v, rows_hbm.at[pl.ds(base, 96)])
    pltpu.sync_copy(chosen_hbm.at[pl.ds(c * TPAD, TPAD)], ch_v)
    lane = lax.iota(jnp.int32, 16)
    tbase = s * 64

    def t_body(t, carry):
        tv = _splat_i32(t)
        val = plsc.load_gather(ch_v, [tv])

        def k_body(k, acc):
            c16 = ch_v[pl.ds(k * 16, 16)]
            m = (c16 == val) & ((lane + k * 16) > tv)
            return acc + plsc.all_reduce_population_count(m)

        later = lax.fori_loop(0, TPAD // 16, k_body, jnp.zeros((16,), jnp.int32))
        outv = jnp.where(later == 0, val, _splat_i32(N))
        plsc.store_scatter(sidx_v, [tv - tbase], outv, mask=(lane == 0))
        return carry

    lax.fori_loop(tbase, tbase + 64, t_body, 0)
    pltpu.sync_copy(sidx_v, sidx_hbm.at[pl.ds(c * TPAD + tbase, 64)])


def _k2_body(rows_ref, delta_ref, ws_ref, out_ref):
    ft = rows_ref[0, 0]
    f1 = rows_ref[0, 1]
    f2 = rows_ref[0, 2]
    interp = ft + delta_ref[0] * (0.5 * (f1 + f2) - ft)
    out_ref[0] = jnp.dot(interp, ws_ref[...], preferred_element_type=jnp.float32)


def _k2(rows, delta_b, w_s):
    return pl.pallas_call(
        _k2_body,
        grid=(L,),
        in_specs=[
            pl.BlockSpec((1, 3, TPAD, D), lambda l: (l, 0, 0, 0)),
            pl.BlockSpec((1, TPAD, D), lambda l: (l, 0, 0)),
            pl.BlockSpec((D, D), lambda l: (0, 0)),
        ],
        out_specs=pl.BlockSpec((1, TPAD, D), lambda l: (l, 0, 0)),
        out_shape=jax.ShapeDtypeStruct((L, TPAD, D), jnp.float32),
    )(rows, delta_b, w_s)


@functools.partial(
    pl.kernel,
    out_type=jax.ShapeDtypeStruct((L * RD, D), jnp.float32),
    mesh=_mesh,
    scratch_types=[
        pltpu.VMEM((CHUNK, D), jnp.float32),
        pltpu.VMEM((64, D), jnp.float32),
        pltpu.VMEM((64,), jnp.int32),
    ],
    compiler_params=_sc_params,
)
def _k3(feat_hbm, nf_hbm, sidx_hbm, fds_hbm, buf_v, nf_v, idx_v):
    c = lax.axis_index("c")
    s = lax.axis_index("s")
    for j in range(5):
        b = jnp.minimum(s * 640 + j * CHUNK, N - CHUNK)
        pltpu.sync_copy(feat_hbm.at[pl.ds(b, CHUNK)], buf_v)
        pltpu.sync_copy(buf_v, fds_hbm.at[pl.ds(c * RD + b, CHUNK)])
    plsc.subcore_barrier()
    base = c * TPAD + s * 64
    pltpu.sync_copy(sidx_hbm.at[pl.ds(base, 64)], idx_v)
    pltpu.sync_copy(nf_hbm.at[pl.ds(base, 64)], nf_v)
    pltpu.sync_copy(nf_v, fds_hbm.at[idx_v])


def _zero_rows(ref, nrows, ncols):
    z = jnp.zeros((16,), jnp.float32)

    def body(i, carry):
        for k in range(ncols // 16):
            ref[i, pl.ds(k * 16, 16)] = z
        return carry

    lax.fori_loop(0, nrows, body, 0)


HR = 10240


def _make_segsum(with_deg):
    acc_rows = N + 16

    out_type = [jax.ShapeDtypeStruct((L * N, D), jnp.float32)]
    scratch = [
        pltpu.VMEM_SHARED((acc_rows, D), jnp.float32),
        pltpu.VMEM((CHUNK,), jnp.int32),
        pltpu.VMEM((CHUNK,), jnp.int32),
        pltpu.VMEM((CHUNK, D), jnp.float32),
        pltpu.SemaphoreType.DMA,
    ]
    if with_deg:
        out_type.append(jax.ShapeDtypeStruct((L * HR,), jnp.float32))
        scratch.append(pltpu.VMEM_SHARED((NS, HR), jnp.float32))
        scratch.append(pltpu.VMEM((HR,), jnp.float32))
        scratch.append(pltpu.VMEM((640,), jnp.float32))
        scratch.append(pltpu.VMEM((640,), jnp.float32))

    def body(x_hbm, src_hbm, dst_hbm, *rest):
        if with_deg:
            (msg_hbm, deg_hbm, acc, sidx_v, didx_v, rows_v, sem,
             dstage, hist, dtmp, daccv) = rest
        else:
            msg_hbm, acc, sidx_v, didx_v, rows_v, sem = rest
        c = lax.axis_index("c")
        s = lax.axis_index("s")
        z16 = jnp.zeros((16,), jnp.float32)
        o16 = jnp.ones((16,), jnp.float32)
        _zero_rows(rows_v, CHUNK, D)
        for j in range(5):
            b = jnp.minimum(s * 640 + j * CHUNK, acc_rows - CHUNK)
            pltpu.sync_copy(rows_v, acc.at[pl.ds(b, CHUNK)])
        if with_deg:
            def zh(i, carry):
                hist[pl.ds(i * 16, 16)] = z16
                return carry

            lax.fori_loop(0, HR // 16, zh, 0)
        plsc.subcore_barrier()

        ebase = c * EPL + s * EPT

        def chunk_body(ch, carry):
            b = ebase + ch * CHUNK
            pltpu.sync_copy(src_hbm.at[pl.ds(b, CHUNK)], sidx_v)
            pltpu.sync_copy(dst_hbm.at[pl.ds(b, CHUNK)], didx_v)
            pltpu.async_copy(x_hbm.at[sidx_v], rows_v, sem).wait()
            pltpu.sync_copy(rows_v, acc.at[didx_v], add=True)
            if with_deg:
                for k in range(CHUNK // 16):
                    plsc.addupdate_scatter(
                        hist, [didx_v[pl.ds(k * 16, 16)]], o16)
            return carry

        lax.fori_loop(0, EPT // CHUNK, chunk_body, 0)
        if with_deg:
            pltpu.sync_copy(hist, dstage.at[s])
        plsc.subcore_barrier()

        for j in range(5):
            b = jnp.minimum(s * 640 + j * CHUNK, N - CHUNK)
            pltpu.sync_copy(acc.at[pl.ds(b, CHUNK)], rows_v)
            pltpu.sync_copy(rows_v, msg_hbm.at[pl.ds(c * N + b, CHUNK)])

        if with_deg:
            sbase = s * 640

            def zb(i, carry):
                daccv[pl.ds(i * 16, 16)] = z16
                return carry

            lax.fori_loop(0, 40, zb, 0)
            for t in range(NS):
                pltpu.sync_copy(dstage.at[t].at[pl.ds(sbase, 640)], dtmp)

                def ab(i, carry):
                    daccv[pl.ds(i * 16, 16)] = (
                        daccv[pl.ds(i * 16, 16)] + dtmp[pl.ds(i * 16, 16)])
                    return carry

                lax.fori_loop(0, 40, ab, 0)
            pltpu.sync_copy(daccv, deg_hbm.at[pl.ds(c * HR + sbase, 640)])

    return pl.kernel(
        body,
        out_type=tuple(out_type) if with_deg else out_type[0],
        mesh=_mesh,
        scratch_types=scratch,
        compiler_params=_sc_params,
    )


_k4 = _make_segsum(with_deg=True)
_k6 = _make_segsum(with_deg=False)


RB = 400


def _make_agg(relu):
    def body(x_ref, msg_ref, deg_ref, w_ref, out_ref):
        d = deg_ref[0, :, 0:1]
        a = (x_ref[0] + msg_ref[0]) / (d + 1.0)
        h = jnp.dot(a, w_ref[...], preferred_element_type=jnp.float32)
        out_ref[0] = jnp.maximum(h, 0.0) if relu else h

    def call(x, msg, deg, w):
        return pl.pallas_call(
            body,
            grid=(L, N // RB),
            in_specs=[
                pl.BlockSpec((1, RB, D), lambda l, r: (l, r, 0)),
                pl.BlockSpec((1, RB, D), lambda l, r: (l, r, 0)),
                pl.BlockSpec((1, RB, 16), lambda l, r: (l, r, 0)),
                pl.BlockSpec((D, H), lambda l, r: (0, 0)),
            ],
            out_specs=pl.BlockSpec((1, RB, H), lambda l, r: (l, r, 0)),
            out_shape=jax.ShapeDtypeStruct((L, N, H), jnp.float32),
        )(x, msg, deg, w)

    return call


_k5 = _make_agg(relu=True)
_k7 = _make_agg(relu=False)


def kernel(feature, adj_new_list, labels, chosen_tail_lists,
           first_neighbor_lists, second_neighbor_lists, center_dict_lists,
           W_s, W1, W2):
    del labels
    chosen = chosen_tail_lists.astype(jnp.int32)
    first = first_neighbor_lists.astype(jnp.int32)
    second = second_neighbor_lists.astype(jnp.int32)
    src = adj_new_list[:, 0, :].astype(jnp.int32)
    dst = adj_new_list[:, 1, :].astype(jnp.int32)

    def pad_t(a, v):
        return jnp.concatenate(
            [a, jnp.full((L, TPAD - T), v, a.dtype)], axis=1)

    cp0 = pad_t(chosen, 0)
    gidx = jnp.stack([cp0, pad_t(first, 0), pad_t(second, 0)],
                     axis=1).reshape(-1)
    chosenN = pad_t(chosen, N).reshape(-1)

    rows, scat_idx = _k1(feature, gidx, chosenN)

    delta_b = jnp.broadcast_to(
        pad_t(center_dict_lists, 0.0)[:, :, None], (L, TPAD, D))
    new_feat = _k2(rows.reshape(L, 3, TPAD, D), delta_b, W_s)

    layer_off = (jnp.arange(L, dtype=jnp.int32) * RD)[:, None]
    sidx_adj = (scat_idx.reshape(L, TPAD) + layer_off).reshape(-1)
    fds = _k3(feature, new_feat.reshape(L * TPAD, D), sidx_adj)

    pad_e = EPL - E
    srcp = jnp.concatenate(
        [src, jnp.zeros((L, pad_e), jnp.int32)], axis=1)
    dstp = jnp.concatenate(
        [dst, jnp.full((L, pad_e), N, jnp.int32)], axis=1)
    src_fds = (srcp + (jnp.arange(L, dtype=jnp.int32) * RD)[:, None]).reshape(-1)
    src_h = (srcp + (jnp.arange(L, dtype=jnp.int32) * N)[:, None]).reshape(-1)
    dstf = dstp.reshape(-1)

    msg1, deg = _k4(fds, src_fds, dstf)
    deg3 = jnp.broadcast_to(deg.reshape(L, HR)[:, :N, None], (L, N, 16))

    x0 = fds.reshape(L, RD, D)[:, :N]
    h = _k5(x0, msg1.reshape(L, N, D), deg3, W1)

    msg2 = _k6(h.reshape(L * N, D), src_h, dstf)
    out = _k7(h, msg2.reshape(L, N, D), deg3, W2)
    return out


# speedup vs baseline: 4.3376x; 1.1179x over previous
import functools

import jax
import jax.numpy as jnp
from jax import lax
from jax.experimental import pallas as pl
from jax.experimental.pallas import tpu as pltpu
from jax.experimental.pallas import tpu_sc as plsc

N = 10000
E = 320000
L = 2
T = 1000
D = 128
H = 128

NC = 2
NS = 16

TPAD = 1024
RD = N + 16
CHUNK = 128
ECHUNK = 80
EPT = 20160
EPL = EPT * NS

_mesh = plsc.VectorSubcoreMesh(core_axis_name="c", subcore_axis_name="s")
_sc_params = pltpu.CompilerParams(needs_layout_passes=False)


def _splat_i32(x):
    return jnp.zeros((16,), jnp.int32) + x


@functools.partial(
    pl.kernel,
    out_type=(
        jax.ShapeDtypeStruct((6 * TPAD, D), jnp.float32),
        jax.ShapeDtypeStruct((L * TPAD,), jnp.int32),
    ),
    mesh=_mesh,
    scratch_types=[
        pltpu.VMEM((96,), jnp.int32),
        pltpu.VMEM((96, D), jnp.float32),
        pltpu.VMEM((TPAD,), jnp.int32),
        pltpu.VMEM((64,), jnp.int32),
        pltpu.SemaphoreType.DMA,
    ],
    compiler_params=_sc_params,
)
def _k1(feat_hbm, gidx_hbm, chosen_hbm, rows_hbm, sidx_hbm,
        idx_v, rows_v, ch_v, sidx_v, sem):
    c = lax.axis_index("c")
    s = lax.axis_index("s")
    wid = c * NS + s
    for j in range(2):
        base = wid * 192 + j * 96
        pltpu.sync_copy(gidx_hbm.at[pl.ds(base, 96)], idx_v)
        pltpu.async_copy(feat_hbm.at[idx_v], rows_v, sem).wait()
        pltpu.sync_copy(rows_v, rows_hbm.at[pl.ds(base, 96)])
    pltpu.sync_copy(chosen_hbm.at[pl.ds(c * TPAD, TPAD)], ch_v)
    lane = lax.iota(jnp.int32, 16)
    tbase = s * 64

    def t_body(t, carry):
        tv = _splat_i32(t)
        val = plsc.load_gather(ch_v, [tv])

        def k_body(k, acc):
            c16 = ch_v[pl.ds(k * 16, 16)]
            m = (c16 == val) & ((lane + k * 16) > tv)
            return acc + plsc.all_reduce_population_count(m)

        later = lax.fori_loop(0, TPAD // 16, k_body, jnp.zeros((16,), jnp.int32))
        outv = jnp.where(later == 0, val, _splat_i32(N))
        plsc.store_scatter(sidx_v, [tv - tbase], outv, mask=(lane == 0))
        return carry

    lax.fori_loop(tbase, tbase + 64, t_body, 0)
    pltpu.sync_copy(sidx_v, sidx_hbm.at[pl.ds(c * TPAD + tbase, 64)])


def _k2_body(rows_ref, delta_ref, ws_ref, out_ref):
    ft = rows_ref[0, 0]
    f1 = rows_ref[0, 1]
    f2 = rows_ref[0, 2]
    interp = ft + delta_ref[0] * (0.5 * (f1 + f2) - ft)
    out_ref[0] = jnp.dot(interp, ws_ref[...], preferred_element_type=jnp.float32)


def _k2(rows, delta_b, w_s):
    return pl.pallas_call(
        _k2_body,
        grid=(L,),
        in_specs=[
            pl.BlockSpec((1, 3, TPAD, D), lambda l: (l, 0, 0, 0)),
            pl.BlockSpec((1, TPAD, D), lambda l: (l, 0, 0)),
            pl.BlockSpec((D, D), lambda l: (0, 0)),
        ],
        out_specs=pl.BlockSpec((1, TPAD, D), lambda l: (l, 0, 0)),
        out_shape=jax.ShapeDtypeStruct((L, TPAD, D), jnp.float32),
    )(rows, delta_b, w_s)


@functools.partial(
    pl.kernel,
    out_type=jax.ShapeDtypeStruct((L * RD, D), jnp.float32),
    mesh=_mesh,
    scratch_types=[
        pltpu.VMEM((CHUNK, D), jnp.float32),
        pltpu.VMEM((64, D), jnp.float32),
        pltpu.VMEM((64,), jnp.int32),
    ],
    compiler_params=_sc_params,
)
def _k3(feat_hbm, nf_hbm, sidx_hbm, fds_hbm, buf_v, nf_v, idx_v):
    c = lax.axis_index("c")
    s = lax.axis_index("s")
    for j in range(5):
        b = jnp.minimum(s * 640 + j * CHUNK, N - CHUNK)
        pltpu.sync_copy(feat_hbm.at[pl.ds(b, CHUNK)], buf_v)
        pltpu.sync_copy(buf_v, fds_hbm.at[pl.ds(c * RD + b, CHUNK)])
    plsc.subcore_barrier()
    base = c * TPAD + s * 64
    pltpu.sync_copy(sidx_hbm.at[pl.ds(base, 64)], idx_v)
    pltpu.sync_copy(nf_hbm.at[pl.ds(base, 64)], nf_v)
    pltpu.sync_copy(nf_v, fds_hbm.at[idx_v])


def _zero_rows(ref, nrows, ncols):
    z = jnp.zeros((16,), jnp.float32)

    def body(i, carry):
        for k in range(ncols // 16):
            ref[i, pl.ds(k * 16, 16)] = z
        return carry

    lax.fori_loop(0, nrows, body, 0)


HR = 10240


def _make_segsum(with_deg, nbuf):
    acc_rows = N + 16

    out_type = [jax.ShapeDtypeStruct((L * N, D), jnp.float32)]
    scratch = [
        pltpu.VMEM_SHARED((acc_rows, D), jnp.float32),
        *[pltpu.VMEM((ECHUNK,), jnp.int32) for _ in range(nbuf)],
        *[pltpu.VMEM((ECHUNK,), jnp.int32) for _ in range(nbuf)],
        *[pltpu.VMEM((ECHUNK, D), jnp.float32) for _ in range(nbuf)],
        *[pltpu.SemaphoreType.DMA for _ in range(nbuf)],
    ]
    if with_deg:
        out_type.append(jax.ShapeDtypeStruct((L * HR,), jnp.float32))
        out_type.append(jax.ShapeDtypeStruct((L * NS * HR,), jnp.float32))
        scratch.append(pltpu.VMEM((HR,), jnp.float32))
        scratch.append(pltpu.VMEM((640,), jnp.float32))
        scratch.append(pltpu.VMEM((640,), jnp.float32))

    def body(x_hbm, src_hbm, dst_hbm, *rest):
        if with_deg:
            msg_hbm, deg_hbm, dstage, acc = rest[0], rest[1], rest[2], rest[3]
            rest = rest[4:]
        else:
            msg_hbm, acc = rest[0], rest[1]
            rest = rest[2:]
        sidx = rest[:nbuf]
        didx = rest[nbuf:2 * nbuf]
        rows = rest[2 * nbuf:3 * nbuf]
        gsem = rest[3 * nbuf:4 * nbuf]
        if with_deg:
            hist, dtmp, daccv = rest[4 * nbuf:]
        rows_v = rows[0]
        c = lax.axis_index("c")
        s = lax.axis_index("s")
        z16 = jnp.zeros((16,), jnp.float32)
        o16 = jnp.ones((16,), jnp.float32)
        _zero_rows(rows_v, ECHUNK, D)
        for j in range(8):
            b = jnp.minimum(s * 640 + j * ECHUNK, acc_rows - ECHUNK)
            pltpu.sync_copy(rows_v, acc.at[pl.ds(b, ECHUNK)])
        if with_deg:
            def zh(i, carry):
                hist[pl.ds(i * 16, 16)] = z16
                return carry

            lax.fori_loop(0, HR // 16, zh, 0)
        plsc.subcore_barrier()

        ebase = c * EPL + s * EPT

        def round_body(g, carry):
            base = ebase + g * (nbuf * ECHUNK)
            descs = []
            for b in range(nbuf):
                pltpu.sync_copy(
                    src_hbm.at[pl.ds(base + b * ECHUNK, ECHUNK)], sidx[b])
                pltpu.sync_copy(
                    dst_hbm.at[pl.ds(base + b * ECHUNK, ECHUNK)], didx[b])
                descs.append(
                    pltpu.async_copy(x_hbm.at[sidx[b]], rows[b], gsem[b]))
            for b in range(nbuf):
                descs[b].wait()
                pltpu.sync_copy(rows[b], acc.at[didx[b]], add=True)
                if with_deg:
                    for k in range(ECHUNK // 16):
                        plsc.addupdate_scatter(
                            hist, [didx[b][pl.ds(k * 16, 16)]], o16)
            return carry

        lax.fori_loop(0, EPT // (nbuf * ECHUNK), round_body, 0)
        if with_deg:
            pltpu.sync_copy(hist, dstage.at[pl.ds((c * NS + s) * HR, HR)])
        plsc.subcore_barrier()

        for j in range(8):
            b = jnp.minimum(s * 640 + j * ECHUNK, N - ECHUNK)
            pltpu.sync_copy(acc.at[pl.ds(b, ECHUNK)], rows_v)
            pltpu.sync_copy(rows_v, msg_hbm.at[pl.ds(c * N + b, ECHUNK)])

        if with_deg:
            sbase = s * 640

            def zb(i, carry):
                daccv[pl.ds(i * 16, 16)] = z16
                return carry

            lax.fori_loop(0, 40, zb, 0)
            for t in range(NS):
                pltpu.sync_copy(
                    dstage.at[pl.ds((c * NS + t) * HR + sbase, 640)], dtmp)

                def ab(i, carry):
                    daccv[pl.ds(i * 16, 16)] = (
                        daccv[pl.ds(i * 16, 16)] + dtmp[pl.ds(i * 16, 16)])
                    return carry

                lax.fori_loop(0, 40, ab, 0)
            pltpu.sync_copy(daccv, deg_hbm.at[pl.ds(c * HR + sbase, 640)])

    return pl.kernel(
        body,
        out_type=tuple(out_type) if with_deg else out_type[0],
        mesh=_mesh,
        scratch_types=scratch,
        compiler_params=_sc_params,
    )


_k4 = _make_segsum(with_deg=True, nbuf=3)
_k6 = _make_segsum(with_deg=False, nbuf=4)


RB = 400


def _make_agg(relu):
    def body(x_ref, msg_ref, deg_ref, w_ref, out_ref):
        d = deg_ref[0, :, 0:1]
        a = (x_ref[0] + msg_ref[0]) / (d + 1.0)
        h = jnp.dot(a, w_ref[...], preferred_element_type=jnp.float32)
        out_ref[0] = jnp.maximum(h, 0.0) if relu else h

    def call(x, msg, deg, w):
        return pl.pallas_call(
            body,
            grid=(L, N // RB),
            in_specs=[
                pl.BlockSpec((1, RB, D), lambda l, r: (l, r, 0)),
                pl.BlockSpec((1, RB, D), lambda l, r: (l, r, 0)),
                pl.BlockSpec((1, RB, 16), lambda l, r: (l, r, 0)),
                pl.BlockSpec((D, H), lambda l, r: (0, 0)),
            ],
            out_specs=pl.BlockSpec((1, RB, H), lambda l, r: (l, r, 0)),
            out_shape=jax.ShapeDtypeStruct((L, N, H), jnp.float32),
        )(x, msg, deg, w)

    return call


_k5 = _make_agg(relu=True)
_k7 = _make_agg(relu=False)


def kernel(feature, adj_new_list, labels, chosen_tail_lists,
           first_neighbor_lists, second_neighbor_lists, center_dict_lists,
           W_s, W1, W2):
    del labels
    chosen = chosen_tail_lists.astype(jnp.int32)
    first = first_neighbor_lists.astype(jnp.int32)
    second = second_neighbor_lists.astype(jnp.int32)
    src = adj_new_list[:, 0, :].astype(jnp.int32)
    dst = adj_new_list[:, 1, :].astype(jnp.int32)

    def pad_t(a, v):
        return jnp.concatenate(
            [a, jnp.full((L, TPAD - T), v, a.dtype)], axis=1)

    cp0 = pad_t(chosen, 0)
    gidx = jnp.stack([cp0, pad_t(first, 0), pad_t(second, 0)],
                     axis=1).reshape(-1)
    chosenN = pad_t(chosen, N).reshape(-1)

    rows, scat_idx = _k1(feature, gidx, chosenN)

    delta_b = jnp.broadcast_to(
        pad_t(center_dict_lists, 0.0)[:, :, None], (L, TPAD, D))
    new_feat = _k2(rows.reshape(L, 3, TPAD, D), delta_b, W_s)

    layer_off = (jnp.arange(L, dtype=jnp.int32) * RD)[:, None]
    sidx_adj = (scat_idx.reshape(L, TPAD) + layer_off).reshape(-1)
    fds = _k3(feature, new_feat.reshape(L * TPAD, D), sidx_adj)

    pad_e = EPL - E
    srcp = jnp.concatenate(
        [src, jnp.zeros((L, pad_e), jnp.int32)], axis=1)
    dstp = jnp.concatenate(
        [dst, jnp.full((L, pad_e), N, jnp.int32)], axis=1)
    src_fds = (srcp + (jnp.arange(L, dtype=jnp.int32) * RD)[:, None]).reshape(-1)
    src_h = (srcp + (jnp.arange(L, dtype=jnp.int32) * N)[:, None]).reshape(-1)
    dstf = dstp.reshape(-1)

    msg1, deg, _ = _k4(fds, src_fds, dstf)
    deg3 = jnp.broadcast_to(deg.reshape(L, HR)[:, :N, None], (L, N, 16))

    x0 = fds.reshape(L, RD, D)[:, :N]
    h = _k5(x0, msg1.reshape(L, N, D), deg3, W1)

    msg2 = _k6(h.reshape(L * N, D), src_h, dstf)
    out = _k7(h, msg2.reshape(L, N, D), deg3, W2)
    return out


# speedup vs baseline: 4.6368x; 1.0690x over previous
import functools

import jax
import jax.numpy as jnp
from jax import lax
from jax.experimental import pallas as pl
from jax.experimental.pallas import tpu as pltpu
from jax.experimental.pallas import tpu_sc as plsc

N = 10000
E = 320000
L = 2
T = 1000
D = 128
H = 128

NC = 2
NS = 16

TPAD = 1024
RD = N + 16
CHUNK = 128
ECHUNK = 80
EPT = 20160
EPL = EPT * NS

_mesh = plsc.VectorSubcoreMesh(core_axis_name="c", subcore_axis_name="s")
_sc_params = pltpu.CompilerParams(needs_layout_passes=False)


def _splat_i32(x):
    return jnp.zeros((16,), jnp.int32) + x


@functools.partial(
    pl.kernel,
    out_type=(
        jax.ShapeDtypeStruct((6 * TPAD, D), jnp.float32),
        jax.ShapeDtypeStruct((L * TPAD,), jnp.int32),
    ),
    mesh=_mesh,
    scratch_types=[
        pltpu.VMEM((96,), jnp.int32),
        pltpu.VMEM((96, D), jnp.float32),
        pltpu.VMEM((TPAD,), jnp.int32),
        pltpu.VMEM((64,), jnp.int32),
        pltpu.SemaphoreType.DMA,
    ],
    compiler_params=_sc_params,
)
def _k1(feat_hbm, gidx_hbm, chosen_hbm, rows_hbm, sidx_hbm,
        idx_v, rows_v, ch_v, sidx_v, sem):
    c = lax.axis_index("c")
    s = lax.axis_index("s")
    wid = c * NS + s
    for j in range(2):
        base = wid * 192 + j * 96
        pltpu.sync_copy(gidx_hbm.at[pl.ds(base, 96)], idx_v)
        pltpu.async_copy(feat_hbm.at[idx_v], rows_v, sem).wait()
        pltpu.sync_copy(rows_v, rows_hbm.at[pl.ds(base, 96)])
    pltpu.sync_copy(chosen_hbm.at[pl.ds(c * TPAD, TPAD)], ch_v)
    lane = lax.iota(jnp.int32, 16)
    tbase = s * 64

    def t_body(t, carry):
        tv = _splat_i32(t)
        val = plsc.load_gather(ch_v, [tv])

        def k_body(k, acc):
            c16 = ch_v[pl.ds(k * 16, 16)]
            m = (c16 == val) & ((lane + k * 16) > tv)
            return acc + plsc.all_reduce_population_count(m)

        later = lax.fori_loop(0, TPAD // 16, k_body, jnp.zeros((16,), jnp.int32))
        outv = jnp.where(later == 0, val, _splat_i32(N))
        plsc.store_scatter(sidx_v, [tv - tbase], outv, mask=(lane == 0))
        return carry

    lax.fori_loop(tbase, tbase + 64, t_body, 0)
    pltpu.sync_copy(sidx_v, sidx_hbm.at[pl.ds(c * TPAD + tbase, 64)])


def _k2_body(rows_ref, delta_ref, ws_ref, out_ref):
    ft = rows_ref[0, 0]
    f1 = rows_ref[0, 1]
    f2 = rows_ref[0, 2]
    interp = ft + delta_ref[0] * (0.5 * (f1 + f2) - ft)
    out_ref[0] = jnp.dot(interp, ws_ref[...], preferred_element_type=jnp.float32)


def _k2(rows, delta_b, w_s):
    return pl.pallas_call(
        _k2_body,
        grid=(L,),
        in_specs=[
            pl.BlockSpec((1, 3, TPAD, D), lambda l: (l, 0, 0, 0)),
            pl.BlockSpec((1, TPAD, D), lambda l: (l, 0, 0)),
            pl.BlockSpec((D, D), lambda l: (0, 0)),
        ],
        out_specs=pl.BlockSpec((1, TPAD, D), lambda l: (l, 0, 0)),
        out_shape=jax.ShapeDtypeStruct((L, TPAD, D), jnp.float32),
    )(rows, delta_b, w_s)


@functools.partial(
    pl.kernel,
    out_type=jax.ShapeDtypeStruct((L * RD, D), jnp.float32),
    mesh=_mesh,
    scratch_types=[
        pltpu.VMEM((CHUNK, D), jnp.float32),
        pltpu.VMEM((64, D), jnp.float32),
        pltpu.VMEM((64,), jnp.int32),
    ],
    compiler_params=_sc_params,
)
def _k3(feat_hbm, nf_hbm, sidx_hbm, fds_hbm, buf_v, nf_v, idx_v):
    c = lax.axis_index("c")
    s = lax.axis_index("s")
    for j in range(5):
        b = jnp.minimum(s * 640 + j * CHUNK, N - CHUNK)
        pltpu.sync_copy(feat_hbm.at[pl.ds(b, CHUNK)], buf_v)
        pltpu.sync_copy(buf_v, fds_hbm.at[pl.ds(c * RD + b, CHUNK)])
    plsc.subcore_barrier()
    base = c * TPAD + s * 64
    pltpu.sync_copy(sidx_hbm.at[pl.ds(base, 64)], idx_v)
    pltpu.sync_copy(nf_hbm.at[pl.ds(base, 64)], nf_v)
    pltpu.sync_copy(nf_v, fds_hbm.at[idx_v])


def _zero_rows(ref, nrows, ncols):
    z = jnp.zeros((16,), jnp.float32)

    def body(i, carry):
        for k in range(ncols // 16):
            ref[i, pl.ds(k * 16, 16)] = z
        return carry

    lax.fori_loop(0, nrows, body, 0)


HR = 10240


def _make_segsum(with_deg, nbuf):
    acc_rows = N + 16

    out_type = [jax.ShapeDtypeStruct((L * N, D), jnp.float32)]
    scratch = [
        pltpu.VMEM_SHARED((acc_rows, D), jnp.float32),
        *[pltpu.VMEM((ECHUNK,), jnp.int32) for _ in range(nbuf)],
        *[pltpu.VMEM((ECHUNK,), jnp.int32) for _ in range(nbuf)],
        *[pltpu.VMEM((ECHUNK, D), jnp.float32) for _ in range(nbuf)],
        *[pltpu.SemaphoreType.DMA for _ in range(2 * nbuf)],
    ]
    if with_deg:
        out_type.append(jax.ShapeDtypeStruct((L * HR,), jnp.float32))
        out_type.append(jax.ShapeDtypeStruct((L * NS * HR,), jnp.float32))
        scratch.append(pltpu.VMEM((HR,), jnp.float32))
        scratch.append(pltpu.VMEM((640,), jnp.float32))
        scratch.append(pltpu.VMEM((640,), jnp.float32))

    def body(x_hbm, src_hbm, dst_hbm, *rest):
        if with_deg:
            msg_hbm, deg_hbm, dstage, acc = rest[0], rest[1], rest[2], rest[3]
            rest = rest[4:]
        else:
            msg_hbm, acc = rest[0], rest[1]
            rest = rest[2:]
        sidx = rest[:nbuf]
        didx = rest[nbuf:2 * nbuf]
        rows = rest[2 * nbuf:3 * nbuf]
        gsem = rest[3 * nbuf:4 * nbuf]
        ssem = rest[4 * nbuf:5 * nbuf]
        if with_deg:
            hist, dtmp, daccv = rest[5 * nbuf:]
        rows_v = rows[0]
        c = lax.axis_index("c")
        s = lax.axis_index("s")
        z16 = jnp.zeros((16,), jnp.float32)
        o16 = jnp.ones((16,), jnp.float32)
        _zero_rows(rows_v, ECHUNK, D)
        for j in range(8):
            b = jnp.minimum(s * 640 + j * ECHUNK, acc_rows - ECHUNK)
            pltpu.sync_copy(rows_v, acc.at[pl.ds(b, ECHUNK)])
        if with_deg:
            def zh(i, carry):
                hist[pl.ds(i * 16, 16)] = z16
                return carry

            lax.fori_loop(0, HR // 16, zh, 0)
        plsc.subcore_barrier()

        ebase = c * EPL + s * EPT

        def emit_round(g, wait_scatter):
            base = ebase + g * (nbuf * ECHUNK)
            descs = []
            for b in range(nbuf):
                if wait_scatter:
                    pltpu.make_async_copy(
                        rows[b], acc.at[didx[b]], ssem[b]).wait()
                pltpu.sync_copy(
                    src_hbm.at[pl.ds(base + b * ECHUNK, ECHUNK)], sidx[b])
                pltpu.sync_copy(
                    dst_hbm.at[pl.ds(base + b * ECHUNK, ECHUNK)], didx[b])
                descs.append(
                    pltpu.async_copy(x_hbm.at[sidx[b]], rows[b], gsem[b]))
            for b in range(nbuf):
                descs[b].wait()
                pltpu.async_copy(
                    rows[b], acc.at[didx[b]], ssem[b], add=True)
                if with_deg:
                    for k in range(ECHUNK // 16):
                        plsc.addupdate_scatter(
                            hist, [didx[b][pl.ds(k * 16, 16)]], o16)

        emit_round(0, wait_scatter=False)

        def round_body(g, carry):
            emit_round(g, wait_scatter=True)
            return carry

        lax.fori_loop(1, EPT // (nbuf * ECHUNK), round_body, 0)
        for b in range(nbuf):
            pltpu.make_async_copy(rows[b], acc.at[didx[b]], ssem[b]).wait()
        if with_deg:
            pltpu.sync_copy(hist, dstage.at[pl.ds((c * NS + s) * HR, HR)])
        plsc.subcore_barrier()

        for j in range(8):
            b = jnp.minimum(s * 640 + j * ECHUNK, N - ECHUNK)
            pltpu.sync_copy(acc.at[pl.ds(b, ECHUNK)], rows_v)
            pltpu.sync_copy(rows_v, msg_hbm.at[pl.ds(c * N + b, ECHUNK)])

        if with_deg:
            sbase = s * 640

            def zb(i, carry):
                daccv[pl.ds(i * 16, 16)] = z16
                return carry

            lax.fori_loop(0, 40, zb, 0)
            for t in range(NS):
                pltpu.sync_copy(
                    dstage.at[pl.ds((c * NS + t) * HR + sbase, 640)], dtmp)

                def ab(i, carry):
                    daccv[pl.ds(i * 16, 16)] = (
                        daccv[pl.ds(i * 16, 16)] + dtmp[pl.ds(i * 16, 16)])
                    return carry

                lax.fori_loop(0, 40, ab, 0)
            pltpu.sync_copy(daccv, deg_hbm.at[pl.ds(c * HR + sbase, 640)])

    return pl.kernel(
        body,
        out_type=tuple(out_type) if with_deg else out_type[0],
        mesh=_mesh,
        scratch_types=scratch,
        compiler_params=_sc_params,
    )


_k4 = _make_segsum(with_deg=True, nbuf=3)
_k6 = _make_segsum(with_deg=False, nbuf=4)


RB = 400


def _make_agg(relu):
    def body(x_ref, msg_ref, deg_ref, w_ref, out_ref):
        d = deg_ref[0, :, 0:1]
        a = (x_ref[0] + msg_ref[0]) / (d + 1.0)
        h = jnp.dot(a, w_ref[...], preferred_element_type=jnp.float32)
        out_ref[0] = jnp.maximum(h, 0.0) if relu else h

    def call(x, msg, deg, w):
        return pl.pallas_call(
            body,
            grid=(L, N // RB),
            in_specs=[
                pl.BlockSpec((1, RB, D), lambda l, r: (l, r, 0)),
                pl.BlockSpec((1, RB, D), lambda l, r: (l, r, 0)),
                pl.BlockSpec((1, RB, 16), lambda l, r: (l, r, 0)),
                pl.BlockSpec((D, H), lambda l, r: (0, 0)),
            ],
            out_specs=pl.BlockSpec((1, RB, H), lambda l, r: (l, r, 0)),
            out_shape=jax.ShapeDtypeStruct((L, N, H), jnp.float32),
        )(x, msg, deg, w)

    return call


_k5 = _make_agg(relu=True)
_k7 = _make_agg(relu=False)


def kernel(feature, adj_new_list, labels, chosen_tail_lists,
           first_neighbor_lists, second_neighbor_lists, center_dict_lists,
           W_s, W1, W2):
    del labels
    chosen = chosen_tail_lists.astype(jnp.int32)
    first = first_neighbor_lists.astype(jnp.int32)
    second = second_neighbor_lists.astype(jnp.int32)
    src = adj_new_list[:, 0, :].astype(jnp.int32)
    dst = adj_new_list[:, 1, :].astype(jnp.int32)

    def pad_t(a, v):
        return jnp.concatenate(
            [a, jnp.full((L, TPAD - T), v, a.dtype)], axis=1)

    cp0 = pad_t(chosen, 0)
    gidx = jnp.stack([cp0, pad_t(first, 0), pad_t(second, 0)],
                     axis=1).reshape(-1)
    chosenN = pad_t(chosen, N).reshape(-1)

    rows, scat_idx = _k1(feature, gidx, chosenN)

    delta_b = jnp.broadcast_to(
        pad_t(center_dict_lists, 0.0)[:, :, None], (L, TPAD, D))
    new_feat = _k2(rows.reshape(L, 3, TPAD, D), delta_b, W_s)

    layer_off = (jnp.arange(L, dtype=jnp.int32) * RD)[:, None]
    sidx_adj = (scat_idx.reshape(L, TPAD) + layer_off).reshape(-1)
    fds = _k3(feature, new_feat.reshape(L * TPAD, D), sidx_adj)

    pad_e = EPL - E
    srcp = jnp.concatenate(
        [src, jnp.zeros((L, pad_e), jnp.int32)], axis=1)
    dstp = jnp.concatenate(
        [dst, jnp.full((L, pad_e), N, jnp.int32)], axis=1)
    src_fds = (srcp + (jnp.arange(L, dtype=jnp.int32) * RD)[:, None]).reshape(-1)
    src_h = (srcp + (jnp.arange(L, dtype=jnp.int32) * N)[:, None]).reshape(-1)
    dstf = dstp.reshape(-1)

    msg1, deg, _ = _k4(fds, src_fds, dstf)
    deg3 = jnp.broadcast_to(deg.reshape(L, HR)[:, :N, None], (L, N, 16))

    x0 = fds.reshape(L, RD, D)[:, :N]
    h = _k5(x0, msg1.reshape(L, N, D), deg3, W1)

    msg2 = _k6(h.reshape(L * N, D), src_h, dstf)
    out = _k7(h, msg2.reshape(L, N, D), deg3, W2)
    return out


# speedup vs baseline: 5.1368x; 1.1078x over previous
import functools

import jax
import jax.numpy as jnp
from jax import lax
from jax.experimental import pallas as pl
from jax.experimental.pallas import tpu as pltpu
from jax.experimental.pallas import tpu_sc as plsc

N = 10000
E = 320000
L = 2
T = 1000
D = 128
H = 128

NC = 2
NS = 16

TPAD = 1024
RD = N + 16
CHUNK = 128
ECHUNK = 80
EPT = 20160
EPL = EPT * NS

_mesh = plsc.VectorSubcoreMesh(core_axis_name="c", subcore_axis_name="s")
_sc_params = pltpu.CompilerParams(needs_layout_passes=False)


def _splat_i32(x):
    return jnp.zeros((16,), jnp.int32) + x


@functools.partial(
    pl.kernel,
    out_type=(
        jax.ShapeDtypeStruct((6 * TPAD, D), jnp.float32),
        jax.ShapeDtypeStruct((L * TPAD,), jnp.int32),
    ),
    mesh=_mesh,
    scratch_types=[
        pltpu.VMEM((96,), jnp.int32),
        pltpu.VMEM((96, D), jnp.float32),
        pltpu.VMEM((TPAD,), jnp.int32),
        pltpu.VMEM((64,), jnp.int32),
        pltpu.SemaphoreType.DMA,
    ],
    compiler_params=_sc_params,
)
def _k1(feat_hbm, gidx_hbm, chosen_hbm, rows_hbm, sidx_hbm,
        idx_v, rows_v, ch_v, sidx_v, sem):
    c = lax.axis_index("c")
    s = lax.axis_index("s")
    wid = c * NS + s
    for j in range(2):
        base = wid * 192 + j * 96
        pltpu.sync_copy(gidx_hbm.at[pl.ds(base, 96)], idx_v)
        pltpu.async_copy(feat_hbm.at[idx_v], rows_v, sem).wait()
        pltpu.sync_copy(rows_v, rows_hbm.at[pl.ds(base, 96)])
    pltpu.sync_copy(chosen_hbm.at[pl.ds(c * TPAD, TPAD)], ch_v)
    lane = lax.iota(jnp.int32, 16)
    tbase = s * 64

    def t_body(t, carry):
        tv = _splat_i32(t)
        val = plsc.load_gather(ch_v, [tv])

        def k_body(k, acc):
            c16 = ch_v[pl.ds(k * 16, 16)]
            m = (c16 == val) & ((lane + k * 16) > tv)
            return acc + plsc.all_reduce_population_count(m)

        later = lax.fori_loop(0, TPAD // 16, k_body, jnp.zeros((16,), jnp.int32))
        outv = jnp.where(later == 0, val, _splat_i32(N))
        plsc.store_scatter(sidx_v, [tv - tbase], outv, mask=(lane == 0))
        return carry

    lax.fori_loop(tbase, tbase + 64, t_body, 0)
    pltpu.sync_copy(sidx_v, sidx_hbm.at[pl.ds(c * TPAD + tbase, 64)])


def _k2_body(rows_ref, delta_ref, ws_ref, out_ref):
    ft = rows_ref[0, 0]
    f1 = rows_ref[0, 1]
    f2 = rows_ref[0, 2]
    interp = ft + delta_ref[0] * (0.5 * (f1 + f2) - ft)
    out_ref[0] = jnp.dot(interp, ws_ref[...], preferred_element_type=jnp.float32)


def _k2(rows, delta_b, w_s):
    return pl.pallas_call(
        _k2_body,
        grid=(L,),
        in_specs=[
            pl.BlockSpec((1, 3, TPAD, D), lambda l: (l, 0, 0, 0)),
            pl.BlockSpec((1, TPAD, D), lambda l: (l, 0, 0)),
            pl.BlockSpec((D, D), lambda l: (0, 0)),
        ],
        out_specs=pl.BlockSpec((1, TPAD, D), lambda l: (l, 0, 0)),
        out_shape=jax.ShapeDtypeStruct((L, TPAD, D), jnp.float32),
    )(rows, delta_b, w_s)


@functools.partial(
    pl.kernel,
    out_type=jax.ShapeDtypeStruct((L * RD, D), jnp.float32),
    mesh=_mesh,
    scratch_types=[
        pltpu.VMEM((CHUNK, D), jnp.float32),
        pltpu.VMEM((64, D), jnp.float32),
        pltpu.VMEM((64,), jnp.int32),
    ],
    compiler_params=_sc_params,
)
def _k3(feat_hbm, nf_hbm, sidx_hbm, fds_hbm, buf_v, nf_v, idx_v):
    c = lax.axis_index("c")
    s = lax.axis_index("s")
    for j in range(5):
        b = jnp.minimum(s * 640 + j * CHUNK, N - CHUNK)
        pltpu.sync_copy(feat_hbm.at[pl.ds(b, CHUNK)], buf_v)
        pltpu.sync_copy(buf_v, fds_hbm.at[pl.ds(c * RD + b, CHUNK)])
    plsc.subcore_barrier()
    base = c * TPAD + s * 64
    pltpu.sync_copy(sidx_hbm.at[pl.ds(base, 64)], idx_v)
    pltpu.sync_copy(nf_hbm.at[pl.ds(base, 64)], nf_v)
    pltpu.sync_copy(nf_v, fds_hbm.at[idx_v])


def _zero_rows(ref, nrows, ncols):
    z = jnp.zeros((16,), jnp.float32)

    def body(i, carry):
        for k in range(ncols // 16):
            ref[i, pl.ds(k * 16, 16)] = z
        return carry

    lax.fori_loop(0, nrows, body, 0)


HR = 10240


def _make_segsum(with_deg, nbuf):
    acc_rows = N + 16

    out_type = [jax.ShapeDtypeStruct((L * N, D), jnp.float32)]
    scratch = [
        pltpu.VMEM_SHARED((acc_rows, D), jnp.float32),
        *[pltpu.VMEM((ECHUNK,), jnp.int32) for _ in range(nbuf)],
        *[pltpu.VMEM((ECHUNK,), jnp.int32) for _ in range(nbuf)],
        *[pltpu.VMEM((ECHUNK, D), jnp.float32) for _ in range(nbuf)],
        *[pltpu.SemaphoreType.DMA for _ in range(3 * nbuf)],
    ]
    if with_deg:
        out_type.append(jax.ShapeDtypeStruct((L * HR,), jnp.float32))
        out_type.append(jax.ShapeDtypeStruct((L * NS * HR,), jnp.float32))
        scratch.append(pltpu.VMEM((HR,), jnp.float32))
        scratch.append(pltpu.VMEM((640,), jnp.float32))
        scratch.append(pltpu.VMEM((640,), jnp.float32))

    def body(x_hbm, src_hbm, dst_hbm, *rest):
        if with_deg:
            msg_hbm, deg_hbm, dstage, acc = rest[0], rest[1], rest[2], rest[3]
            rest = rest[4:]
        else:
            msg_hbm, acc = rest[0], rest[1]
            rest = rest[2:]
        sidx = rest[:nbuf]
        didx = rest[nbuf:2 * nbuf]
        rows = rest[2 * nbuf:3 * nbuf]
        gsem = rest[3 * nbuf:4 * nbuf]
        ssem = rest[4 * nbuf:5 * nbuf]
        isem = rest[5 * nbuf:6 * nbuf]
        if with_deg:
            hist, dtmp, daccv = rest[6 * nbuf:]
        rows_v = rows[0]
        c = lax.axis_index("c")
        s = lax.axis_index("s")
        z16 = jnp.zeros((16,), jnp.float32)
        o16 = jnp.ones((16,), jnp.float32)
        _zero_rows(rows_v, ECHUNK, D)
        zdescs = []
        for j in range(8):
            b = jnp.minimum(s * 640 + j * ECHUNK, acc_rows - ECHUNK)
            zdescs.append(
                pltpu.async_copy(rows_v, acc.at[pl.ds(b, ECHUNK)], gsem[0]))
        for zd in zdescs:
            zd.wait()
        if with_deg:
            def zh(i, carry):
                hist[pl.ds(i * 16, 16)] = z16
                return carry

            lax.fori_loop(0, HR // 16, zh, 0)
        plsc.subcore_barrier()

        ebase = c * EPL + s * EPT

        def emit_round(g, wait_scatter):
            base = ebase + g * (nbuf * ECHUNK)
            idescs = []
            for b in range(nbuf):
                if wait_scatter:
                    pltpu.make_async_copy(
                        rows[b], acc.at[didx[b]], ssem[b]).wait()
                idescs.append((
                    pltpu.async_copy(
                        src_hbm.at[pl.ds(base + b * ECHUNK, ECHUNK)],
                        sidx[b], isem[b]),
                    pltpu.async_copy(
                        dst_hbm.at[pl.ds(base + b * ECHUNK, ECHUNK)],
                        didx[b], isem[b])))
            descs = []
            for b in range(nbuf):
                idescs[b][0].wait()
                idescs[b][1].wait()
                descs.append(
                    pltpu.async_copy(x_hbm.at[sidx[b]], rows[b], gsem[b]))
            for b in range(nbuf):
                descs[b].wait()
                pltpu.async_copy(
                    rows[b], acc.at[didx[b]], ssem[b], add=True)
                if with_deg:
                    for k in range(ECHUNK // 16):
                        plsc.addupdate_scatter(
                            hist, [didx[b][pl.ds(k * 16, 16)]], o16)

        emit_round(0, wait_scatter=False)

        def round_body(g, carry):
            emit_round(g, wait_scatter=True)
            return carry

        lax.fori_loop(1, EPT // (nbuf * ECHUNK), round_body, 0)
        for b in range(nbuf):
            pltpu.make_async_copy(rows[b], acc.at[didx[b]], ssem[b]).wait()
        if with_deg:
            pltpu.sync_copy(hist, dstage.at[pl.ds((c * NS + s) * HR, HR)])
        plsc.subcore_barrier()

        hdescs = {}
        for j in range(8):
            bb = j % nbuf
            if j >= nbuf:
                hdescs[bb].wait()
            b = jnp.minimum(s * 640 + j * ECHUNK, N - ECHUNK)
            pltpu.sync_copy(acc.at[pl.ds(b, ECHUNK)], rows[bb])
            hdescs[bb] = pltpu.async_copy(
                rows[bb], msg_hbm.at[pl.ds(c * N + b, ECHUNK)], ssem[bb])
        for bb in range(nbuf):
            hdescs[bb].wait()

        if with_deg:
            sbase = s * 640

            def zb(i, carry):
                daccv[pl.ds(i * 16, 16)] = z16
                return carry

            lax.fori_loop(0, 40, zb, 0)
            for t in range(NS):
                pltpu.sync_copy(
                    dstage.at[pl.ds((c * NS + t) * HR + sbase, 640)], dtmp)

                def ab(i, carry):
                    daccv[pl.ds(i * 16, 16)] = (
                        daccv[pl.ds(i * 16, 16)] + dtmp[pl.ds(i * 16, 16)])
                    return carry

                lax.fori_loop(0, 40, ab, 0)
            pltpu.sync_copy(daccv, deg_hbm.at[pl.ds(c * HR + sbase, 640)])

    return pl.kernel(
        body,
        out_type=tuple(out_type) if with_deg else out_type[0],
        mesh=_mesh,
        scratch_types=scratch,
        compiler_params=_sc_params,
    )


_k4 = _make_segsum(with_deg=True, nbuf=3)
_k6 = _make_segsum(with_deg=False, nbuf=4)


RB = 400


def _make_agg(relu):
    def body(x_ref, msg_ref, deg_ref, w_ref, out_ref):
        d = deg_ref[0, :, 0:1]
        a = (x_ref[0] + msg_ref[0]) / (d + 1.0)
        h = jnp.dot(a, w_ref[...], preferred_element_type=jnp.float32)
        out_ref[0] = jnp.maximum(h, 0.0) if relu else h

    def call(x, msg, deg, w):
        return pl.pallas_call(
            body,
            grid=(L, N // RB),
            in_specs=[
                pl.BlockSpec((1, RB, D), lambda l, r: (l, r, 0)),
                pl.BlockSpec((1, RB, D), lambda l, r: (l, r, 0)),
                pl.BlockSpec((1, RB, 16), lambda l, r: (l, r, 0)),
                pl.BlockSpec((D, H), lambda l, r: (0, 0)),
            ],
            out_specs=pl.BlockSpec((1, RB, H), lambda l, r: (l, r, 0)),
            out_shape=jax.ShapeDtypeStruct((L, N, H), jnp.float32),
        )(x, msg, deg, w)

    return call


_k5 = _make_agg(relu=True)
_k7 = _make_agg(relu=False)


def kernel(feature, adj_new_list, labels, chosen_tail_lists,
           first_neighbor_lists, second_neighbor_lists, center_dict_lists,
           W_s, W1, W2):
    del labels
    chosen = chosen_tail_lists.astype(jnp.int32)
    first = first_neighbor_lists.astype(jnp.int32)
    second = second_neighbor_lists.astype(jnp.int32)
    src = adj_new_list[:, 0, :].astype(jnp.int32)
    dst = adj_new_list[:, 1, :].astype(jnp.int32)

    def pad_t(a, v):
        return jnp.concatenate(
            [a, jnp.full((L, TPAD - T), v, a.dtype)], axis=1)

    cp0 = pad_t(chosen, 0)
    gidx = jnp.stack([cp0, pad_t(first, 0), pad_t(second, 0)],
                     axis=1).reshape(-1)
    chosenN = pad_t(chosen, N).reshape(-1)

    rows, scat_idx = _k1(feature, gidx, chosenN)

    delta_b = jnp.broadcast_to(
        pad_t(center_dict_lists, 0.0)[:, :, None], (L, TPAD, D))
    new_feat = _k2(rows.reshape(L, 3, TPAD, D), delta_b, W_s)

    layer_off = (jnp.arange(L, dtype=jnp.int32) * RD)[:, None]
    sidx_adj = (scat_idx.reshape(L, TPAD) + layer_off).reshape(-1)
    fds = _k3(feature, new_feat.reshape(L * TPAD, D), sidx_adj)

    pad_e = EPL - E
    srcp = jnp.concatenate(
        [src, jnp.zeros((L, pad_e), jnp.int32)], axis=1)
    dstp = jnp.concatenate(
        [dst, jnp.full((L, pad_e), N, jnp.int32)], axis=1)
    src_fds = (srcp + (jnp.arange(L, dtype=jnp.int32) * RD)[:, None]).reshape(-1)
    src_h = (srcp + (jnp.arange(L, dtype=jnp.int32) * N)[:, None]).reshape(-1)
    dstf = dstp.reshape(-1)

    msg1, deg, _ = _k4(fds, src_fds, dstf)
    deg3 = jnp.broadcast_to(deg.reshape(L, HR)[:, :N, None], (L, N, 16))

    x0 = fds.reshape(L, RD, D)[:, :N]
    h = _k5(x0, msg1.reshape(L, N, D), deg3, W1)

    msg2 = _k6(h.reshape(L * N, D), src_h, dstf)
    out = _k7(h, msg2.reshape(L, N, D), deg3, W2)
    return out
